# Initial kernel scaffold; baseline (speedup 1.0000x reference)
import jax
import jax.numpy as jnp
from jax.experimental import pallas as pl


def kernel(x, edge_index, edge_type, graph_ids, rel_labels, head_ids, tail_ids, rel_w, W, W_self, rel_emb, fc_w, fc_b):
    raise NotImplementedError("write your pallas kernel here")


# speedup vs baseline: 2.0358x; 2.0358x over previous
import functools

import jax
import jax.numpy as jnp
from jax import lax
from jax.experimental import pallas as pl
from jax.experimental.pallas import tpu as pltpu
from jax.experimental.pallas import tpu_sc as plsc

N = 10000
E = 160000
D = 256
L = 3
R = 64
B = 100

N_PAD = 10240
E_PAD = 163840
NSUB = 16
EPW = E_PAD // NSUB
NPW = N_PAD // NSUB
K = 64
NCHUNK = EPW // K
NT = N_PAD // 512

_mesh = plsc.VectorSubcoreMesh(core_axis_name="c", subcore_axis_name="s",
                               num_cores=2, num_subcores=NSUB)


@functools.partial(
    pl.kernel,
    out_type=jax.ShapeDtypeStruct((N_PAD, 16), jnp.float32),
    mesh=_mesh,
    scratch_types=[
        pltpu.VMEM((K,), jnp.int32),
        pltpu.VMEM((K, 128), jnp.float32),
        pltpu.VMEM((K, 16), jnp.float32),
        pltpu.VMEM_SHARED((N_PAD, 128), jnp.float32),
    ],
)
def _deg_kernel(dst_hbm, inv_hbm, dstbuf, onesbuf, obuf, acc):
    c = lax.axis_index("c")
    s = lax.axis_index("s")
    zeros16 = jnp.zeros((16,), jnp.float32)
    ones16 = jnp.ones((16,), jnp.float32)

    @pl.when(c == 0)
    def _():
        def zfill(r, _):
            for k in range(8):
                onesbuf[r, pl.ds(16 * k, 16)] = zeros16
            return 0

        lax.fori_loop(0, K, zfill, 0)
        for j in range(NPW // K):
            pltpu.sync_copy(onesbuf, acc.at[pl.ds(s * NPW + j * K, K), :])

        def fill(r, _):
            for k in range(8):
                onesbuf[r, pl.ds(16 * k, 16)] = ones16
            return 0

        lax.fori_loop(0, K, fill, 0)

    plsc.subcore_barrier()

    @pl.when(c == 0)
    def _():
        def chunk(i, _):
            pltpu.sync_copy(dst_hbm.at[pl.ds(s * EPW + i * K, K)], dstbuf)
            pltpu.sync_copy(onesbuf, acc.at[dstbuf], add=True)
            return 0

        lax.fori_loop(0, EPW // K, chunk, 0)

    plsc.subcore_barrier()

    @pl.when(c == 0)
    def _():
        for j in range(NPW // K):
            rj = s * NPW + j * K
            pltpu.sync_copy(acc.at[pl.ds(rj, K), :], onesbuf)

            def ibody(r, _):
                v = onesbuf[r, pl.ds(0, 16)]
                obuf[r, :] = 1.0 / jnp.maximum(v, 1.0)
                return 0

            lax.fori_loop(0, K, ibody, 0)
            pltpu.sync_copy(obuf, inv_hbm.at[pl.ds(rj, K), :])


@functools.partial(
    pl.kernel,
    out_type=jax.ShapeDtypeStruct((2, N_PAD, 128), jnp.float32),
    mesh=_mesh,
    scratch_types=[
        pltpu.VMEM((K,), jnp.int32),
        pltpu.VMEM((K,), jnp.int32),
        pltpu.VMEM((K,), jnp.int32),
        pltpu.VMEM((K, 128), jnp.float32),
        pltpu.VMEM((K, 128), jnp.float32),
        pltpu.VMEM((K, 16), jnp.float32),
        pltpu.VMEM_SHARED((N_PAD, 128), jnp.float32),
        pltpu.SemaphoreType.DMA,
        pltpu.SemaphoreType.DMA,
    ],
)
def _edge_kernel(srcA, srcB, dst, typeA, typeB, h2, relw2, invdeg, out,
                 srcbuf, dstbuf, typebuf, hrows, gaterows, invbuf, acc,
                 gsem, gsem2):
    c = lax.axis_index("c")
    s = lax.axis_index("s")
    r0 = s * NPW

    zeros16 = jnp.zeros((16,), jnp.float32)

    def zrow(r, _):
        for k in range(8):
            hrows[r, pl.ds(16 * k, 16)] = zeros16
        return 0

    lax.fori_loop(0, K, zrow, 0)
    for j in range(NPW // K):
        pltpu.sync_copy(hrows, acc.at[pl.ds(r0 + j * K, K), :])
    plsc.subcore_barrier()

    def chunk(i, _):
        e0 = s * EPW + i * K

        @pl.when(c == 0)
        def _():
            pltpu.sync_copy(srcA.at[pl.ds(e0, K)], srcbuf)
            pltpu.sync_copy(typeA.at[pl.ds(e0, K)], typebuf)

        @pl.when(c == 1)
        def _():
            pltpu.sync_copy(srcB.at[pl.ds(e0, K)], srcbuf)
            pltpu.sync_copy(typeB.at[pl.ds(e0, K)], typebuf)

        pltpu.sync_copy(dst.at[pl.ds(e0, K)], dstbuf)
        g1 = pltpu.async_copy(h2.at[srcbuf], hrows, gsem)
        g2 = pltpu.async_copy(relw2.at[typebuf], gaterows, gsem2)
        g1.wait()
        g2.wait()

        def mul_row(r, _):
            for k in range(8):
                sl = pl.ds(16 * k, 16)
                hrows[r, sl] = hrows[r, sl] * gaterows[r, sl]
            return 0

        lax.fori_loop(0, K, mul_row, 0)
        pltpu.sync_copy(hrows, acc.at[dstbuf], add=True)
        return 0

    lax.fori_loop(0, NCHUNK, chunk, 0)
    plsc.subcore_barrier()

    for j in range(NPW // K):
        rj = r0 + j * K
        pltpu.sync_copy(acc.at[pl.ds(rj, K), :], hrows)
        pltpu.sync_copy(invdeg.at[pl.ds(rj, K), :], invbuf)

        def scale_row(r, _):
            iv = invbuf[r, :]
            for k in range(8):
                sl = pl.ds(16 * k, 16)
                hrows[r, sl] = hrows[r, sl] * iv
            return 0

        lax.fori_loop(0, K, scale_row, 0)

        @pl.when(c == 0)
        def _():
            pltpu.sync_copy(hrows, out.at[0, pl.ds(rj, K), :])

        @pl.when(c == 1)
        def _():
            pltpu.sync_copy(hrows, out.at[1, pl.ds(rj, K), :])


def _layer_body(a0, a1, h, w0, w1, ws, o):
    acc = jnp.dot(a0[...], w0[...], preferred_element_type=jnp.float32)
    acc += jnp.dot(a1[...], w1[...], preferred_element_type=jnp.float32)
    acc += jnp.dot(h[...], ws[...], preferred_element_type=jnp.float32)
    o[...] = jnp.maximum(acc, 0.0)


_LAYER_SPECS = dict(
    grid=(NT,),
    in_specs=[
        pl.BlockSpec((512, 128), lambda i: (i, 0)),
        pl.BlockSpec((512, 128), lambda i: (i, 0)),
        pl.BlockSpec((512, 256), lambda i: (i, 0)),
        pl.BlockSpec((128, 256), lambda i: (0, 0)),
        pl.BlockSpec((128, 256), lambda i: (0, 0)),
        pl.BlockSpec((256, 256), lambda i: (0, 0)),
    ],
    out_specs=pl.BlockSpec((512, 256), lambda i: (i, 0)),
    out_shape=jax.ShapeDtypeStruct((N_PAD, 256), jnp.float32),
)

_layer_call = pl.pallas_call(_layer_body, **_LAYER_SPECS)


def _tail_body(h1, h2, h3, wp1, wp2, wp3, gid, hid, tid, rlab, rele, fcr,
               o, acc):
    i = pl.program_id(0)

    @pl.when(i == 0)
    def _():
        acc[...] = jnp.zeros_like(acc)

    f32 = jnp.float32
    P = jnp.dot(h1[...], wp1[...], preferred_element_type=f32)
    P += jnp.dot(h2[...], wp2[...], preferred_element_type=f32)
    P += jnp.dot(h3[...], wp3[...], preferred_element_type=f32)
    col = lax.broadcasted_iota(jnp.int32, (512, 128), 1)
    P = P + jnp.where(col == 3, 1.0, 0.0)

    gid_v = gid[...].reshape(1, 512)
    bidx = lax.broadcasted_iota(jnp.int32, (128, 512), 0)
    n_glob = i * 512 + lax.broadcasted_iota(jnp.int32, (128, 512), 1)
    hid_v = hid[...].reshape(128, 1)
    tid_v = tid[...].reshape(128, 1)

    Mg = (gid_v == bidx).astype(f32)
    Mh = (hid_v == n_glob).astype(f32)
    Mt = (tid_v == n_glob).astype(f32)

    Pg = P * jnp.where((col == 0) | (col == 3), 1.0, 0.0)
    Ph = P * jnp.where(col == 1, 1.0, 0.0)
    Pt = P * jnp.where(col == 2, 1.0, 0.0)

    acc[...] += (jnp.dot(Mg, Pg, preferred_element_type=f32)
                 + jnp.dot(Mh, Ph, preferred_element_type=f32)
                 + jnp.dot(Mt, Pt, preferred_element_type=f32))

    @pl.when(i == NT - 1)
    def _():
        RP = jnp.dot(rele[...], fcr[...], preferred_element_type=f32)
        rlab_v = rlab[...].reshape(128, 1)
        Mr = (rlab_v == lax.broadcasted_iota(jnp.int32, (128, R), 1)).astype(f32)
        relt = jnp.dot(Mr, RP, preferred_element_type=f32)
        A = acc[...]
        col128 = lax.broadcasted_iota(jnp.int32, (128, 128), 1)

        def colsum(M, j):
            return jnp.sum(M * (col128 == j).astype(f32), axis=1, keepdims=True)

        g = colsum(A, 0)
        cntv = jnp.maximum(colsum(A, 3), 1.0)
        res = g / cntv + colsum(A, 1) + colsum(A, 2) + colsum(relt, 0)
        o[...] = jnp.broadcast_to(res, (128, 128))


_TAIL_SPECS = dict(
    grid=(NT,),
    in_specs=[
        pl.BlockSpec((512, 256), lambda i: (i, 0)),
        pl.BlockSpec((512, 256), lambda i: (i, 0)),
        pl.BlockSpec((512, 256), lambda i: (i, 0)),
        pl.BlockSpec((256, 128), lambda i: (0, 0)),
        pl.BlockSpec((256, 128), lambda i: (0, 0)),
        pl.BlockSpec((256, 128), lambda i: (0, 0)),
        pl.BlockSpec((1, 1, 512), lambda i: (i, 0, 0)),
        pl.BlockSpec((1, 1, 128), lambda i: (0, 0, 0)),
        pl.BlockSpec((1, 1, 128), lambda i: (0, 0, 0)),
        pl.BlockSpec((1, 1, 128), lambda i: (0, 0, 0)),
        pl.BlockSpec((R, 256), lambda i: (0, 0)),
        pl.BlockSpec((256, 128), lambda i: (0, 0)),
    ],
    out_specs=pl.BlockSpec((128, 128), lambda i: (0, 0)),
    out_shape=jax.ShapeDtypeStruct((128, 128), jnp.float32),
    scratch_shapes=[pltpu.VMEM((128, 128), jnp.float32)],
)

_tail_call = pl.pallas_call(_tail_body, **_TAIL_SPECS)


def kernel(x, edge_index, edge_type, graph_ids, rel_labels, head_ids,
           tail_ids, rel_w, W, W_self, rel_emb, fc_w, fc_b):
    src = edge_index[0]
    dst = edge_index[1]
    pad_e = E_PAD - E
    zpad = jnp.zeros((pad_e,), jnp.int32)
    srcA = jnp.concatenate([src * 2, zpad])
    srcB = srcA + 1
    typeA = jnp.concatenate([edge_type * 2, zpad])
    typeB = typeA + 1
    dstp = jnp.concatenate([dst, jnp.full((pad_e,), N, jnp.int32)])

    invdeg = _deg_kernel(dstp)

    h = jnp.concatenate([x, jnp.zeros((N_PAD - N, D), jnp.float32)], axis=0)
    hs = []
    for l in range(L):
        aggs = _edge_kernel(srcA, srcB, dstp, typeA, typeB,
                            h.reshape(2 * N_PAD, 128),
                            rel_w[l].reshape(2 * R, 128), invdeg)
        h = _layer_call(aggs[0], aggs[1], h,
                        W[l][:128, :], W[l][128:, :], W_self[l])
        hs.append(h)

    fw = fc_w[:, 0]
    wps = []
    for l in range(L):
        wp = jnp.stack([fw[256 * l:256 * (l + 1)],
                        fw[768 + 256 * l:768 + 256 * (l + 1)],
                        fw[1536 + 256 * l:1536 + 256 * (l + 1)]], axis=1)
        wps.append(jnp.pad(wp, ((0, 0), (0, 125))))
    fcr = jnp.pad(fc_w[2304:2560], ((0, 0), (0, 127)))

    gidp = jnp.concatenate(
        [graph_ids, jnp.full((N_PAD - N,), -1, jnp.int32)]).reshape(NT, 1, 512)
    hidp = jnp.pad(head_ids, (0, 128 - B), constant_values=-1).reshape(1, 1, 128)
    tidp = jnp.pad(tail_ids, (0, 128 - B), constant_values=-1).reshape(1, 1, 128)
    rlabp = jnp.pad(rel_labels, (0, 128 - B), constant_values=-1).reshape(1, 1, 128)

    tail = _tail_call(hs[0], hs[1], hs[2], wps[0], wps[1], wps[2],
                      gidp, hidp, tidp, rlabp, rel_emb, fcr)
    return tail[:B, 0:1] + fc_b


# speedup vs baseline: 2.5878x; 1.2711x over previous
import functools

import jax
import jax.numpy as jnp
from jax import lax
from jax.experimental import pallas as pl
from jax.experimental.pallas import tpu as pltpu
from jax.experimental.pallas import tpu_sc as plsc

N = 10000
E = 160000
D = 256
L = 3
R = 64
B = 100

N_PAD = 10240
E_PAD = 163840
NSUB = 16
EPW = E_PAD // NSUB
NPW = N_PAD // NSUB
K = 64
NCHUNK = EPW // K
NT = N_PAD // 512

_mesh = plsc.VectorSubcoreMesh(core_axis_name="c", subcore_axis_name="s",
                               num_cores=2, num_subcores=NSUB)


@functools.partial(
    pl.kernel,
    out_type=jax.ShapeDtypeStruct((N_PAD, 16), jnp.float32),
    mesh=_mesh,
    scratch_types=[
        pltpu.VMEM((K,), jnp.int32),
        pltpu.VMEM((K, 128), jnp.float32),
        pltpu.VMEM((K, 16), jnp.float32),
        pltpu.VMEM_SHARED((N_PAD, 128), jnp.float32),
    ],
)
def _deg_kernel(dst_hbm, inv_hbm, dstbuf, onesbuf, obuf, acc):
    c = lax.axis_index("c")
    s = lax.axis_index("s")
    zeros16 = jnp.zeros((16,), jnp.float32)
    ones16 = jnp.ones((16,), jnp.float32)

    @pl.when(c == 0)
    def _():
        def zfill(r, _):
            for k in range(8):
                onesbuf[r, pl.ds(16 * k, 16)] = zeros16
            return 0

        lax.fori_loop(0, K, zfill, 0)
        for j in range(NPW // K):
            pltpu.sync_copy(onesbuf, acc.at[pl.ds(s * NPW + j * K, K), :])

        def fill(r, _):
            for k in range(8):
                onesbuf[r, pl.ds(16 * k, 16)] = ones16
            return 0

        lax.fori_loop(0, K, fill, 0)

    plsc.subcore_barrier()

    @pl.when(c == 0)
    def _():
        def chunk(i, _):
            pltpu.sync_copy(dst_hbm.at[pl.ds(s * EPW + i * K, K)], dstbuf)
            pltpu.sync_copy(onesbuf, acc.at[dstbuf], add=True)
            return 0

        lax.fori_loop(0, EPW // K, chunk, 0)

    plsc.subcore_barrier()

    @pl.when(c == 0)
    def _():
        for j in range(NPW // K):
            rj = s * NPW + j * K
            pltpu.sync_copy(acc.at[pl.ds(rj, K), :], onesbuf)

            def ibody(r, _):
                v = onesbuf[r, pl.ds(0, 16)]
                obuf[r, :] = 1.0 / jnp.maximum(v, 1.0)
                return 0

            lax.fori_loop(0, K, ibody, 0)
            pltpu.sync_copy(obuf, inv_hbm.at[pl.ds(rj, K), :])


SCH = 16
NSUPER = NCHUNK // SCH
ROWS_PW = EPW // K
N_ACC = 10112
NPW_A = N_ACC // NSUB
_BLOCKS = [64] * 9 + [56]


def _mul_rows(a, b):
    def mul_row(r, _):
        for k in range(8):
            sl = pl.ds(16 * k, 16)
            a[r, sl] = a[r, sl] * b[r, sl]
        return 0

    lax.fori_loop(0, K, mul_row, 0)


@functools.partial(
    pl.kernel,
    out_type=jax.ShapeDtypeStruct((2, N_ACC, 128), jnp.float32),
    mesh=_mesh,
    scratch_types=[
        pltpu.VMEM((SCH, K), jnp.int32),
        pltpu.VMEM((SCH, K), jnp.int32),
        pltpu.VMEM((SCH, K), jnp.int32),
        pltpu.VMEM((K, 128), jnp.float32),
        pltpu.VMEM((K, 128), jnp.float32),
        pltpu.VMEM((K, 128), jnp.float32),
        pltpu.VMEM((K, 128), jnp.float32),
        pltpu.VMEM((K, 16), jnp.float32),
        pltpu.VMEM_SHARED((N_ACC, 128), jnp.float32),
        pltpu.SemaphoreType.DMA,
        pltpu.SemaphoreType.DMA,
        pltpu.SemaphoreType.DMA,
        pltpu.SemaphoreType.DMA,
    ],
)
def _edge_kernel(srcA, srcB, dst2, typeA, typeB, h2, relw2, invdeg, out,
                 srcbuf, dstbuf, typebuf, hrows0, hrows1, gaterows0,
                 gaterows1, invbuf, acc, gsem0, gsem1, ssem0, ssem1):
    c = lax.axis_index("c")
    s = lax.axis_index("s")
    r0 = s * NPW_A

    zeros16 = jnp.zeros((16,), jnp.float32)

    def zrow(r, _):
        for k in range(8):
            hrows0[r, pl.ds(16 * k, 16)] = zeros16
        return 0

    lax.fori_loop(0, K, zrow, 0)
    rq = 0
    for bs in _BLOCKS:
        pltpu.sync_copy(hrows0.at[pl.ds(0, bs), :],
                        acc.at[pl.ds(r0 + rq, bs), :])
        rq += bs
    plsc.subcore_barrier()

    def superchunk(sidx, _):
        base = s * ROWS_PW + sidx * SCH

        @pl.when(c == 0)
        def _():
            pltpu.sync_copy(srcA.at[pl.ds(base, SCH), :], srcbuf)
            pltpu.sync_copy(typeA.at[pl.ds(base, SCH), :], typebuf)

        @pl.when(c == 1)
        def _():
            pltpu.sync_copy(srcB.at[pl.ds(base, SCH), :], srcbuf)
            pltpu.sync_copy(typeB.at[pl.ds(base, SCH), :], typebuf)

        pltpu.sync_copy(dst2.at[pl.ds(base, SCH), :], dstbuf)

        def pair(q, _):
            i0 = 2 * q
            i1 = 2 * q + 1
            g0a = pltpu.async_copy(h2.at[srcbuf.at[i0]], hrows0, gsem0)
            g0b = pltpu.async_copy(relw2.at[typebuf.at[i0]], gaterows0, gsem0)
            g1a = pltpu.async_copy(h2.at[srcbuf.at[i1]], hrows1, gsem1)
            g1b = pltpu.async_copy(relw2.at[typebuf.at[i1]], gaterows1, gsem1)
            g0a.wait()
            g0b.wait()
            _mul_rows(hrows0, gaterows0)
            s0 = pltpu.async_copy(hrows0, acc.at[dstbuf.at[i0]], ssem0,
                                  add=True)
            g1a.wait()
            g1b.wait()
            _mul_rows(hrows1, gaterows1)
            s1 = pltpu.async_copy(hrows1, acc.at[dstbuf.at[i1]], ssem1,
                                  add=True)
            s0.wait()
            s1.wait()
            return 0

        lax.fori_loop(0, SCH // 2, pair, 0)
        return 0

    lax.fori_loop(0, NSUPER, superchunk, 0)
    plsc.subcore_barrier()

    rq = 0
    for bs in _BLOCKS:
        rj = r0 + rq
        rq += bs
        pltpu.sync_copy(acc.at[pl.ds(rj, bs), :], hrows0.at[pl.ds(0, bs), :])
        pltpu.sync_copy(invdeg.at[pl.ds(rj, bs), :],
                        invbuf.at[pl.ds(0, bs), :])

        def scale_row(r, _):
            iv = invbuf[r, :]
            for k in range(8):
                sl = pl.ds(16 * k, 16)
                hrows0[r, sl] = hrows0[r, sl] * iv
            return 0

        lax.fori_loop(0, bs, scale_row, 0)

        @pl.when(c == 0)
        def _():
            pltpu.sync_copy(hrows0.at[pl.ds(0, bs), :],
                            out.at[0, pl.ds(rj, bs), :])

        @pl.when(c == 1)
        def _():
            pltpu.sync_copy(hrows0.at[pl.ds(0, bs), :],
                            out.at[1, pl.ds(rj, bs), :])


def _layer_body(a0, a1, h, w0, w1, ws, o):
    acc = jnp.dot(a0[...], w0[...], preferred_element_type=jnp.float32)
    acc += jnp.dot(a1[...], w1[...], preferred_element_type=jnp.float32)
    acc += jnp.dot(h[...], ws[...], preferred_element_type=jnp.float32)
    o[...] = jnp.maximum(acc, 0.0)


_LAYER_SPECS = dict(
    grid=(NT,),
    in_specs=[
        pl.BlockSpec((512, 128), lambda i: (i, 0)),
        pl.BlockSpec((512, 128), lambda i: (i, 0)),
        pl.BlockSpec((512, 256), lambda i: (i, 0)),
        pl.BlockSpec((128, 256), lambda i: (0, 0)),
        pl.BlockSpec((128, 256), lambda i: (0, 0)),
        pl.BlockSpec((256, 256), lambda i: (0, 0)),
    ],
    out_specs=pl.BlockSpec((512, 256), lambda i: (i, 0)),
    out_shape=jax.ShapeDtypeStruct((N_PAD, 256), jnp.float32),
)

_layer_call = pl.pallas_call(_layer_body, **_LAYER_SPECS)


def _tail_body(h1, h2, h3, wp1, wp2, wp3, gid, hid, tid, rlab, rele, fcr,
               o, acc):
    i = pl.program_id(0)

    @pl.when(i == 0)
    def _():
        acc[...] = jnp.zeros_like(acc)

    f32 = jnp.float32
    P = jnp.dot(h1[...], wp1[...], preferred_element_type=f32)
    P += jnp.dot(h2[...], wp2[...], preferred_element_type=f32)
    P += jnp.dot(h3[...], wp3[...], preferred_element_type=f32)
    col = lax.broadcasted_iota(jnp.int32, (512, 128), 1)
    P = P + jnp.where(col == 3, 1.0, 0.0)

    gid_v = gid[...].reshape(1, 512)
    bidx = lax.broadcasted_iota(jnp.int32, (128, 512), 0)
    n_glob = i * 512 + lax.broadcasted_iota(jnp.int32, (128, 512), 1)
    hid_v = hid[...].reshape(128, 1)
    tid_v = tid[...].reshape(128, 1)

    Mg = (gid_v == bidx).astype(f32)
    Mh = (hid_v == n_glob).astype(f32)
    Mt = (tid_v == n_glob).astype(f32)

    Pg = P * jnp.where((col == 0) | (col == 3), 1.0, 0.0)
    Ph = P * jnp.where(col == 1, 1.0, 0.0)
    Pt = P * jnp.where(col == 2, 1.0, 0.0)

    acc[...] += (jnp.dot(Mg, Pg, preferred_element_type=f32)
                 + jnp.dot(Mh, Ph, preferred_element_type=f32)
                 + jnp.dot(Mt, Pt, preferred_element_type=f32))

    @pl.when(i == NT - 1)
    def _():
        RP = jnp.dot(rele[...], fcr[...], preferred_element_type=f32)
        rlab_v = rlab[...].reshape(128, 1)
        Mr = (rlab_v == lax.broadcasted_iota(jnp.int32, (128, R), 1)).astype(f32)
        relt = jnp.dot(Mr, RP, preferred_element_type=f32)
        A = acc[...]
        col128 = lax.broadcasted_iota(jnp.int32, (128, 128), 1)

        def colsum(M, j):
            return jnp.sum(M * (col128 == j).astype(f32), axis=1, keepdims=True)

        g = colsum(A, 0)
        cntv = jnp.maximum(colsum(A, 3), 1.0)
        res = g / cntv + colsum(A, 1) + colsum(A, 2) + colsum(relt, 0)
        o[...] = jnp.broadcast_to(res, (128, 128))


_TAIL_SPECS = dict(
    grid=(NT,),
    in_specs=[
        pl.BlockSpec((512, 256), lambda i: (i, 0)),
        pl.BlockSpec((512, 256), lambda i: (i, 0)),
        pl.BlockSpec((512, 256), lambda i: (i, 0)),
        pl.BlockSpec((256, 128), lambda i: (0, 0)),
        pl.BlockSpec((256, 128), lambda i: (0, 0)),
        pl.BlockSpec((256, 128), lambda i: (0, 0)),
        pl.BlockSpec((1, 1, 512), lambda i: (i, 0, 0)),
        pl.BlockSpec((1, 1, 128), lambda i: (0, 0, 0)),
        pl.BlockSpec((1, 1, 128), lambda i: (0, 0, 0)),
        pl.BlockSpec((1, 1, 128), lambda i: (0, 0, 0)),
        pl.BlockSpec((R, 256), lambda i: (0, 0)),
        pl.BlockSpec((256, 128), lambda i: (0, 0)),
    ],
    out_specs=pl.BlockSpec((128, 128), lambda i: (0, 0)),
    out_shape=jax.ShapeDtypeStruct((128, 128), jnp.float32),
    scratch_shapes=[pltpu.VMEM((128, 128), jnp.float32)],
)

_tail_call = pl.pallas_call(_tail_body, **_TAIL_SPECS)


def kernel(x, edge_index, edge_type, graph_ids, rel_labels, head_ids,
           tail_ids, rel_w, W, W_self, rel_emb, fc_w, fc_b):
    src = edge_index[0]
    dst = edge_index[1]
    pad_e = E_PAD - E
    zpad = jnp.zeros((pad_e,), jnp.int32)
    srcA = jnp.concatenate([src * 2, zpad])
    srcB = srcA + 1
    typeA = jnp.concatenate([edge_type * 2, zpad])
    typeB = typeA + 1
    dstp = jnp.concatenate([dst, jnp.full((pad_e,), N, jnp.int32)])

    invdeg = _deg_kernel(dstp)

    h = jnp.concatenate([x, jnp.zeros((N_PAD - N, D), jnp.float32)], axis=0)
    hs = []
    for l in range(L):
        aggs = _edge_kernel(srcA.reshape(-1, K), srcB.reshape(-1, K),
                            dstp.reshape(-1, K), typeA.reshape(-1, K),
                            typeB.reshape(-1, K),
                            h.reshape(2 * N_PAD, 128),
                            rel_w[l].reshape(2 * R, 128), invdeg)
        a0 = jnp.pad(aggs[0], ((0, N_PAD - N_ACC), (0, 0)))
        a1 = jnp.pad(aggs[1], ((0, N_PAD - N_ACC), (0, 0)))
        h = _layer_call(a0, a1, h,
                        W[l][:128, :], W[l][128:, :], W_self[l])
        hs.append(h)

    fw = fc_w[:, 0]
    wps = []
    for l in range(L):
        wp = jnp.stack([fw[256 * l:256 * (l + 1)],
                        fw[768 + 256 * l:768 + 256 * (l + 1)],
                        fw[1536 + 256 * l:1536 + 256 * (l + 1)]], axis=1)
        wps.append(jnp.pad(wp, ((0, 0), (0, 125))))
    fcr = jnp.pad(fc_w[2304:2560], ((0, 0), (0, 127)))

    gidp = jnp.concatenate(
        [graph_ids, jnp.full((N_PAD - N,), -1, jnp.int32)]).reshape(NT, 1, 512)
    hidp = jnp.pad(head_ids, (0, 128 - B), constant_values=-1).reshape(1, 1, 128)
    tidp = jnp.pad(tail_ids, (0, 128 - B), constant_values=-1).reshape(1, 1, 128)
    rlabp = jnp.pad(rel_labels, (0, 128 - B), constant_values=-1).reshape(1, 1, 128)

    tail = _tail_call(hs[0], hs[1], hs[2], wps[0], wps[1], wps[2],
                      gidp, hidp, tidp, rlabp, rel_emb, fcr)
    return tail[:B, 0:1] + fc_b
